# baseline (speedup 1.0000x reference)
import jax
import jax.numpy as jnp
from jax.experimental import pallas as pl
from jax.experimental.pallas import tpu as pltpu

_TILE = 1000


def _mlp_outer(x_hbm, w1_hbm, b1_hbm, w2_hbm, b2_hbm, o_hbm,
               w1_v, b1_v, w2_v, b2_v, sem):
    copies = (
        pltpu.make_async_copy(w1_hbm, w1_v, sem.at[0]),
        pltpu.make_async_copy(b1_hbm, b1_v, sem.at[1]),
        pltpu.make_async_copy(w2_hbm, w2_v, sem.at[2]),
        pltpu.make_async_copy(b2_hbm, b2_v, sem.at[3]),
    )
    for c in copies:
        c.start()
    for c in copies:
        c.wait()

    n, in_ch = x_hbm.shape
    ncls = o_hbm.shape[1]

    def inner(x_blk, o_blk):
        h = jnp.dot(x_blk[:], w1_v[:], preferred_element_type=jnp.float32)
        h = jnp.maximum(h + b1_v[:][None, :], 0.0)
        o_blk[:] = (
            jnp.dot(h, w2_v[:], preferred_element_type=jnp.float32)
            + b2_v[:][None, :]
        )

    pltpu.emit_pipeline(
        inner,
        grid=(n // _TILE,),
        in_specs=[pl.BlockSpec((_TILE, in_ch), lambda i: (i, 0))],
        out_specs=[pl.BlockSpec((_TILE, ncls), lambda i: (i, 0))],
    )(x_hbm, o_hbm)


def kernel(x, edge_index, W1, b1, W2, b2):
    n, in_ch = x.shape
    hid = W1.shape[1]
    ncls = W2.shape[1]
    return pl.pallas_call(
        _mlp_outer,
        in_specs=[pl.BlockSpec(memory_space=pltpu.MemorySpace.HBM)] * 5,
        out_specs=pl.BlockSpec(memory_space=pltpu.MemorySpace.HBM),
        out_shape=jax.ShapeDtypeStruct((n, ncls), jnp.float32),
        scratch_shapes=[
            pltpu.VMEM((in_ch, hid), jnp.float32),
            pltpu.VMEM((hid,), jnp.float32),
            pltpu.VMEM((hid, ncls), jnp.float32),
            pltpu.VMEM((ncls,), jnp.float32),
            pltpu.SemaphoreType.DMA((4,)),
        ],
    )(x, W1, b1, W2, b2)


# speedup vs baseline: 2.1901x; 2.1901x over previous
import jax
import jax.numpy as jnp
from jax.experimental import pallas as pl
from jax.experimental.pallas import tpu as pltpu

_TILE = 1024


def _mlp_kernel(x_ref, w1t_ref, b1_ref, w2t_ref, b2_ref, o_ref):
    hT = jax.lax.dot_general(
        w1t_ref[:], x_ref[:], (((1,), (1,)), ((), ())),
        preferred_element_type=jnp.float32,
    )
    b1c = b1_ref[:][None, :].T
    hT = jnp.maximum(hT + b1c, 0.0)
    oT = jnp.dot(w2t_ref[:], hT, preferred_element_type=jnp.float32)
    b2c = b2_ref[:][None, :].T
    o_ref[:] = oT + b2c


def kernel(x, edge_index, W1, b1, W2, b2):
    n, in_ch = x.shape
    hid = W1.shape[1]
    ncls = W2.shape[1]
    grid = (n + _TILE - 1) // _TILE
    outT = pl.pallas_call(
        _mlp_kernel,
        grid=(grid,),
        in_specs=[
            pl.BlockSpec((_TILE, in_ch), lambda i: (i, 0)),
            pl.BlockSpec((hid, in_ch), lambda i: (0, 0)),
            pl.BlockSpec((hid,), lambda i: (0,)),
            pl.BlockSpec((ncls, hid), lambda i: (0, 0)),
            pl.BlockSpec((ncls,), lambda i: (0,)),
        ],
        out_specs=pl.BlockSpec((ncls, _TILE), lambda i: (0, i)),
        out_shape=jax.ShapeDtypeStruct((ncls, n), jnp.float32),
        compiler_params=pltpu.CompilerParams(
            dimension_semantics=("parallel",),
        ),
    )(x, W1.T, b1, W2.T, b2)
    return outT.T


# speedup vs baseline: 3.0856x; 1.4089x over previous
import jax
import jax.numpy as jnp
from jax.experimental import pallas as pl
from jax.experimental.pallas import tpu as pltpu

_TILE = 2048


def _mlp_kernel(x_ref, w1t_ref, b1_ref, w2t_ref, b2_ref, o_ref):
    hT = jax.lax.dot_general(
        w1t_ref[:], x_ref[:], (((1,), (1,)), ((), ())),
        preferred_element_type=jnp.float32,
    )
    b1c = b1_ref[:][None, :].T
    hT = jnp.maximum(hT + b1c, 0.0)
    oT = jnp.dot(w2t_ref[:], hT, preferred_element_type=jnp.float32)
    b2c = b2_ref[:][None, :].T
    o_ref[:] = oT + b2c


def kernel(x, edge_index, W1, b1, W2, b2):
    n, in_ch = x.shape
    hid = W1.shape[1]
    ncls = W2.shape[1]
    grid = (n + _TILE - 1) // _TILE
    outT = pl.pallas_call(
        _mlp_kernel,
        grid=(grid,),
        in_specs=[
            pl.BlockSpec((_TILE, in_ch), lambda i: (i, 0)),
            pl.BlockSpec((hid, in_ch), lambda i: (0, 0)),
            pl.BlockSpec((hid,), lambda i: (0,)),
            pl.BlockSpec((ncls, hid), lambda i: (0, 0)),
            pl.BlockSpec((ncls,), lambda i: (0,)),
        ],
        out_specs=pl.BlockSpec((ncls, _TILE), lambda i: (0, i)),
        out_shape=jax.ShapeDtypeStruct((ncls, n), jnp.float32),
        compiler_params=pltpu.CompilerParams(
            dimension_semantics=("parallel",),
        ),
    )(x, W1.T, b1, W2.T, b2)
    return outT.T


# speedup vs baseline: 4.4512x; 1.4425x over previous
import jax
import jax.numpy as jnp
from jax.experimental import pallas as pl
from jax.experimental.pallas import tpu as pltpu

_TILE = 5120


def _mlp_kernel(x_ref, w1t_ref, b1_ref, w2t_ref, b2_ref, o_ref):
    hT = jax.lax.dot_general(
        w1t_ref[:], x_ref[:], (((1,), (1,)), ((), ())),
        preferred_element_type=jnp.float32,
    )
    b1c = b1_ref[:][None, :].T
    hT = jnp.maximum(hT + b1c, 0.0)
    oT = jnp.dot(w2t_ref[:], hT, preferred_element_type=jnp.float32)
    b2c = b2_ref[:][None, :].T
    o_ref[:] = oT + b2c


def kernel(x, edge_index, W1, b1, W2, b2):
    n, in_ch = x.shape
    hid = W1.shape[1]
    ncls = W2.shape[1]
    grid = (n + _TILE - 1) // _TILE
    outT = pl.pallas_call(
        _mlp_kernel,
        grid=(grid,),
        in_specs=[
            pl.BlockSpec((_TILE, in_ch), lambda i: (i, 0)),
            pl.BlockSpec(memory_space=pltpu.MemorySpace.VMEM),
            pl.BlockSpec(memory_space=pltpu.MemorySpace.VMEM),
            pl.BlockSpec(memory_space=pltpu.MemorySpace.VMEM),
            pl.BlockSpec(memory_space=pltpu.MemorySpace.VMEM),
        ],
        out_specs=pl.BlockSpec((ncls, _TILE), lambda i: (0, i)),
        out_shape=jax.ShapeDtypeStruct((ncls, n), jnp.float32),
        compiler_params=pltpu.CompilerParams(
            dimension_semantics=("parallel",),
        ),
    )(x, W1.T, b1, W2.T, b2)
    return outT.T
